# baseline (speedup 1.0000x reference)
import jax
import jax.numpy as jnp
from jax import lax
from jax.experimental import pallas as pl
from jax.experimental.pallas import tpu as pltpu
from jax.experimental.pallas import tpu_sc as plsc

VOCAB = 1000000
SEQ_LEN = 200
EMBED = 32
PAD = 128
LROW = 128
BATCH = 4096

NC, NS = 2, 16
NW = NC * NS
B_PER_W = BATCH // NW
CB = 8
NCHUNKS = B_PER_W // CB
SPLIT = 128
REST = SEQ_LEN - SPLIT

_MESH = plsc.VectorSubcoreMesh(
    core_axis_name="c", subcore_axis_name="s", num_cores=NC, num_subcores=NS
)


def _wid():
    return lax.axis_index("s") * NC + lax.axis_index("c")


def _idx_prep_body(idx_hbm, out_hbm, sem):
    base = _wid() * B_PER_W

    def transfers(j):
        b0 = pl.multiple_of(base + CB * j, CB)
        o0 = pl.multiple_of(2 * b0, CB)
        for c in range(2):
            yield (
                idx_hbm.at[pl.ds(b0, CB), pl.ds(c * SPLIT, SPLIT)],
                out_hbm.at[pl.ds(o0 + c * CB, CB)],
            )

    def fire(j, _):
        for src, dst in transfers(j):
            pltpu.async_copy(src, dst, sem)
        return 0

    lax.fori_loop(0, NCHUNKS, fire, 0)

    def drain(j, _):
        for src, dst in transfers(j):
            pltpu.make_async_copy(src, dst, sem).wait()
        return 0

    lax.fori_loop(0, NCHUNKS, drain, 0)


def _body(tok_hbm, idx_hbm, pos_hbm, out_hbm, idx_v, rows_v, pos_v, sem):
    base_b = _wid() * B_PER_W

    pltpu.sync_copy(pos_hbm, pos_v)

    def chunk_body(ci, _):
        b0 = pl.multiple_of(base_b + ci * CB, CB)
        pltpu.sync_copy(idx_hbm.at[pl.ds(2 * b0, 2 * CB)], idx_v)
        for r in range(CB):
            pltpu.async_copy(
                tok_hbm.at[idx_v.at[r]],
                rows_v.at[r, pl.ds(0, SPLIT)],
                sem,
            )
            pltpu.async_copy(
                tok_hbm.at[idx_v.at[CB + r, pl.ds(0, REST)]],
                rows_v.at[r, pl.ds(SPLIT, REST)],
                sem,
            )
        for r in range(CB):
            pltpu.make_async_copy(
                tok_hbm.at[idx_v.at[r]],
                rows_v.at[r, pl.ds(0, SPLIT)],
                sem,
            ).wait()
            pltpu.make_async_copy(
                tok_hbm.at[idx_v.at[CB + r, pl.ds(0, REST)]],
                rows_v.at[r, pl.ds(SPLIT, REST)],
                sem,
            ).wait()

        def add_body(l, _):
            p0 = pos_v[l, 0:16]
            p1 = pos_v[l, 16:32]
            for b in range(CB):
                rows_v[b, l, 0:16] = rows_v[b, l, 0:16] + p0
                rows_v[b, l, 16:32] = rows_v[b, l, 16:32] + p1
            return 0

        lax.fori_loop(0, SEQ_LEN, add_body, 0)

        pltpu.sync_copy(
            rows_v, out_hbm.at[pl.ds(b0, CB), slice(None), pl.ds(0, EMBED)]
        )
        return 0

    lax.fori_loop(0, NCHUNKS, chunk_body, 0)


@jax.jit
def _run(tok, idx, pos):
    idxp = jnp.pad(idx, ((0, 0), (0, 2 * SPLIT - SEQ_LEN)))
    idx2 = pl.kernel(
        _idx_prep_body,
        out_type=jax.ShapeDtypeStruct((2 * BATCH, LROW), jnp.int32),
        mesh=_MESH,
        scratch_types=[pltpu.SemaphoreType.DMA],
    )(idxp)
    out = pl.kernel(
        _body,
        out_type=jax.ShapeDtypeStruct((BATCH, SEQ_LEN, PAD), jnp.float32),
        mesh=_MESH,
        scratch_types=[
            pltpu.VMEM((2 * CB, LROW), jnp.int32),
            pltpu.VMEM((CB, SEQ_LEN, EMBED), jnp.float32),
            pltpu.VMEM((SEQ_LEN, EMBED), jnp.float32),
            pltpu.SemaphoreType.DMA,
        ],
        compiler_params=pltpu.CompilerParams(use_tc_tiling_on_sc=False),
    )(tok, idx2, pos)
    return out[..., :EMBED]


def kernel(inputs, token_table, pos_table):
    return _run(token_table, inputs, pos_table)


# speedup vs baseline: 1.0003x; 1.0003x over previous
import jax
import jax.numpy as jnp
from jax import lax
from jax.experimental import pallas as pl
from jax.experimental.pallas import tpu as pltpu
from jax.experimental.pallas import tpu_sc as plsc

VOCAB = 1000000
SEQ_LEN = 200
EMBED = 32
PAD = 128
LROW = 128
BATCH = 4096

NC, NS = 2, 16
NW = NC * NS
B_PER_W = BATCH // NW
CB = 8
NPANELS = BATCH // CB
NCHUNKS = B_PER_W // CB
SPLIT = 128
REST = SEQ_LEN - SPLIT

_MESH = plsc.VectorSubcoreMesh(
    core_axis_name="c", subcore_axis_name="s", num_cores=NC, num_subcores=NS
)


def _body(tok_hbm, idx_hbm, pos_hbm, out_hbm, idx_v, rows_v, pos_v, sem):
    wid = lax.axis_index("s") * NC + lax.axis_index("c")
    base_g = wid * NCHUNKS

    pltpu.sync_copy(pos_hbm, pos_v)

    def chunk_body(ci, _):
        g = base_g + ci
        b0 = pl.multiple_of(g * CB, CB)
        pltpu.sync_copy(idx_hbm.at[g], idx_v)
        for r in range(CB):
            pltpu.async_copy(
                tok_hbm.at[idx_v.at[0, r]],
                rows_v.at[r, pl.ds(0, SPLIT)],
                sem,
            )
            pltpu.async_copy(
                tok_hbm.at[idx_v.at[1, r, pl.ds(0, REST)]],
                rows_v.at[r, pl.ds(SPLIT, REST)],
                sem,
            )
        for r in range(CB):
            pltpu.make_async_copy(
                tok_hbm.at[idx_v.at[0, r]],
                rows_v.at[r, pl.ds(0, SPLIT)],
                sem,
            ).wait()
            pltpu.make_async_copy(
                tok_hbm.at[idx_v.at[1, r, pl.ds(0, REST)]],
                rows_v.at[r, pl.ds(SPLIT, REST)],
                sem,
            ).wait()

        def add_body(l, _):
            p0 = pos_v[l, 0:16]
            p1 = pos_v[l, 16:32]
            for b in range(CB):
                rows_v[b, l, 0:16] = rows_v[b, l, 0:16] + p0
                rows_v[b, l, 16:32] = rows_v[b, l, 16:32] + p1
            return 0

        lax.fori_loop(0, SEQ_LEN, add_body, 0)

        pltpu.sync_copy(
            rows_v, out_hbm.at[pl.ds(b0, CB), slice(None), pl.ds(0, EMBED)]
        )
        return 0

    lax.fori_loop(0, NCHUNKS, chunk_body, 0)


@jax.jit
def _run(tok, idx, pos):
    idx4 = (
        jnp.pad(idx, ((0, 0), (0, 2 * SPLIT - SEQ_LEN)))
        .reshape(NPANELS, CB, 2, LROW)
        .transpose(0, 2, 1, 3)
    )
    out = pl.kernel(
        _body,
        out_type=jax.ShapeDtypeStruct((BATCH, SEQ_LEN, PAD), jnp.float32),
        mesh=_MESH,
        scratch_types=[
            pltpu.VMEM((2, CB, LROW), jnp.int32),
            pltpu.VMEM((CB, SEQ_LEN, EMBED), jnp.float32),
            pltpu.VMEM((SEQ_LEN, EMBED), jnp.float32),
            pltpu.SemaphoreType.DMA,
        ],
        compiler_params=pltpu.CompilerParams(use_tc_tiling_on_sc=False),
    )(tok, idx4, pos)
    return out[..., :EMBED]


def kernel(inputs, token_table, pos_table):
    return _run(token_table, inputs, pos_table)
